# Initial kernel scaffold; baseline (speedup 1.0000x reference)
import jax
import jax.numpy as jnp
from jax.experimental import pallas as pl


def kernel(x, table):
    raise NotImplementedError("write your pallas kernel here")


# speedup vs baseline: 2.4901x; 2.4901x over previous
import functools

import jax
import jax.numpy as jnp
from jax import lax
from jax.experimental import pallas as pl
from jax.experimental.pallas import tpu as pltpu, tpu_sc as plsc


def _make_gather(V, D, B):
    info = plsc.get_sparse_core_info()
    NC, NS = info.num_cores, info.num_subcores
    NW = NC * NS
    assert B % NW == 0
    per_w = B // NW
    C = 2048
    assert per_w % C == 0
    n_iter = per_w // C

    mesh = plsc.VectorSubcoreMesh(core_axis_name="c", subcore_axis_name="s")

    @functools.partial(
        pl.kernel,
        mesh=mesh,
        out_type=jax.ShapeDtypeStruct((B, D), jnp.float32),
        scratch_types=[
            pltpu.VMEM((C,), jnp.int32),
            pltpu.VMEM((C, D), jnp.float32),
            pltpu.SemaphoreType.DMA,
        ],
        compiler_params=pltpu.CompilerParams(use_tc_tiling_on_sc=False),
    )
    def k(table_hbm, idx_hbm, out_hbm, idx_v, rows_v, sem):
        wid = lax.axis_index("s") * NC + lax.axis_index("c")
        w_base = wid * per_w

        def body(i, carry):
            base = w_base + i * C
            pltpu.sync_copy(idx_hbm.at[pl.ds(base, C)], idx_v)
            pltpu.async_copy(table_hbm.at[idx_v], rows_v, sem).wait()
            pltpu.sync_copy(rows_v, out_hbm.at[pl.ds(base, C)])
            return carry

        lax.fori_loop(0, n_iter, body, 0)

    return k


def kernel(x, table):
    Bt, H = x.shape
    V, D = table.shape
    B = Bt * H
    flat_idx = x.reshape(B)
    out = _make_gather(V, D, B)(table, flat_idx)
    return out.reshape(Bt, H, D)


# speedup vs baseline: 2.5675x; 1.0311x over previous
import functools

import jax
import jax.numpy as jnp
from jax import lax
from jax.experimental import pallas as pl
from jax.experimental.pallas import tpu as pltpu, tpu_sc as plsc


def _make_gather(V, D, B):
    info = plsc.get_sparse_core_info()
    NC, NS = info.num_cores, info.num_subcores
    NW = NC * NS
    assert B % NW == 0
    per_w = B // NW
    C = 2048
    NBUF = 2
    assert per_w % C == 0
    n_iter = per_w // C
    assert n_iter % NBUF == 0 and n_iter >= 2 * NBUF

    mesh = plsc.VectorSubcoreMesh(core_axis_name="c", subcore_axis_name="s")

    @functools.partial(
        pl.kernel,
        mesh=mesh,
        out_type=jax.ShapeDtypeStruct((B, D), jnp.float32),
        scratch_types=[
            [pltpu.VMEM((C,), jnp.int32) for _ in range(NBUF)],
            [pltpu.VMEM((C, D), jnp.float32) for _ in range(NBUF)],
            [pltpu.SemaphoreType.DMA for _ in range(NBUF)],
            [pltpu.SemaphoreType.DMA for _ in range(NBUF)],
        ],
        compiler_params=pltpu.CompilerParams(use_tc_tiling_on_sc=False),
    )
    def k(table_hbm, idx_hbm, out_hbm, idx_v, rows_v, sem_g, sem_s):
        wid = lax.axis_index("s") * NC + lax.axis_index("c")
        w_base = wid * per_w

        def gather_start(it, b):
            base = w_base + it * C
            pltpu.sync_copy(idx_hbm.at[pl.ds(base, C)], idx_v[b])
            return pltpu.async_copy(table_hbm.at[idx_v[b]], rows_v[b], sem_g[b])

        def store_start(it, b):
            base = w_base + it * C
            return pltpu.async_copy(rows_v[b], out_hbm.at[pl.ds(base, C)], sem_s[b])

        def wait_gather(b):
            pltpu.make_async_copy(table_hbm.at[idx_v[b]], rows_v[b], sem_g[b]).wait()

        def wait_store(it, b):
            base = w_base + it * C
            pltpu.make_async_copy(rows_v[b], out_hbm.at[pl.ds(base, C)], sem_s[b]).wait()

        for b in range(NBUF):
            gather_start(b, b)

        def body(o, carry):
            it0 = o * NBUF
            for b in range(NBUF):
                it = it0 + b
                wait_gather(b)
                store_start(it, b)

                @pl.when(it + NBUF < n_iter)
                def _():
                    wait_store(it, b)
                    gather_start(it + NBUF, b)

            return carry

        lax.fori_loop(0, n_iter // NBUF, body, 0)

        for b in range(NBUF):
            wait_store(n_iter - NBUF + b, b)

    return k


def kernel(x, table):
    Bt, H = x.shape
    V, D = table.shape
    B = Bt * H
    flat_idx = x.reshape(B)
    out = _make_gather(V, D, B)(table, flat_idx)
    return out.reshape(Bt, H, D)


# speedup vs baseline: 2.5690x; 1.0006x over previous
import functools

import jax
import jax.numpy as jnp
from jax import lax
from jax.experimental import pallas as pl
from jax.experimental.pallas import tpu as pltpu, tpu_sc as plsc


def _make_gather(V, D, B):
    info = plsc.get_sparse_core_info()
    NC, NS = info.num_cores, info.num_subcores
    NW = NC * NS
    assert B % NW == 0
    per_w = B // NW
    C = 3200
    NBUF = 2
    assert per_w % C == 0
    n_iter = per_w // C
    assert n_iter % NBUF == 0 and n_iter >= 2 * NBUF

    mesh = plsc.VectorSubcoreMesh(core_axis_name="c", subcore_axis_name="s")

    @functools.partial(
        pl.kernel,
        mesh=mesh,
        out_type=jax.ShapeDtypeStruct((B, D), jnp.float32),
        scratch_types=[
            [pltpu.VMEM((C,), jnp.int32) for _ in range(NBUF)],
            [pltpu.VMEM((C, D), jnp.float32) for _ in range(NBUF)],
            [pltpu.SemaphoreType.DMA for _ in range(NBUF)],
            [pltpu.SemaphoreType.DMA for _ in range(NBUF)],
        ],
        compiler_params=pltpu.CompilerParams(use_tc_tiling_on_sc=False),
    )
    def k(table_hbm, idx_hbm, out_hbm, idx_v, rows_v, sem_g, sem_s):
        wid = lax.axis_index("s") * NC + lax.axis_index("c")
        w_base = wid * per_w

        def gather_start(it, b):
            base = w_base + it * C
            pltpu.sync_copy(idx_hbm.at[pl.ds(base, C)], idx_v[b])
            return pltpu.async_copy(table_hbm.at[idx_v[b]], rows_v[b], sem_g[b])

        def store_start(it, b):
            base = w_base + it * C
            return pltpu.async_copy(rows_v[b], out_hbm.at[pl.ds(base, C)], sem_s[b])

        def wait_gather(b):
            pltpu.make_async_copy(table_hbm.at[idx_v[b]], rows_v[b], sem_g[b]).wait()

        def wait_store(it, b):
            base = w_base + it * C
            pltpu.make_async_copy(rows_v[b], out_hbm.at[pl.ds(base, C)], sem_s[b]).wait()

        for b in range(NBUF):
            gather_start(b, b)

        def body(o, carry):
            it0 = o * NBUF
            for b in range(NBUF):
                it = it0 + b
                wait_gather(b)
                store_start(it, b)

                @pl.when(it + NBUF < n_iter)
                def _():
                    wait_store(it, b)
                    gather_start(it + NBUF, b)

            return carry

        lax.fori_loop(0, n_iter // NBUF, body, 0)

        for b in range(NBUF):
            wait_store(n_iter - NBUF + b, b)

    return k


def kernel(x, table):
    Bt, H = x.shape
    V, D = table.shape
    B = Bt * H
    flat_idx = x.reshape(B)
    out = _make_gather(V, D, B)(table, flat_idx)
    return out.reshape(Bt, H, D)
